# baseline (speedup 1.0000x reference)
import functools

import jax
import jax.numpy as jnp
from jax import lax
from jax.experimental import pallas as pl
from jax.experimental.pallas import tpu as pltpu
from jax.experimental.pallas import tpu_sc as plsc

B, C, H, W = 8, 768, 32, 32
HW = H * W
NPIX = B * HW
NREF = 4
LANES = 16
NC, NS = 2, 16
NW = NC * NS
PPW = NPIX // NW
GROUP = 8
GPW = PPW // GROUP
NCHUNK = C // LANES
SCALE = 1.0 / float(C) ** 0.5


def _tc_body(x_ref, wq_ref, wk_ref, wv_ref, wo_ref, bq_ref, bk_ref, bv_ref,
             bo_ref, v_ref, s_ref, gidx_ref):
    b = pl.program_id(0)
    xb = x_ref[...]
    q = jnp.dot(xb, wq_ref[...], preferred_element_type=jnp.float32) + bq_ref[...]
    k = jnp.dot(xb, wk_ref[...], preferred_element_type=jnp.float32) + bk_ref[...]
    v_ref[...] = jnp.dot(xb, wv_ref[...], preferred_element_type=jnp.float32) + bv_ref[...]
    s_ref[...] = lax.dot_general(q, k, (((1,), (1,)), ((), ())),
                                 preferred_element_type=jnp.float32)
    off = jnp.dot(q, wo_ref[...], preferred_element_type=jnp.float32) + bo_ref[...]
    p = lax.broadcasted_iota(jnp.int32, (HW, 1), 0)
    ypix = (p // W).astype(jnp.float32)
    xpix = (p % W).astype(jnp.float32)
    cols = []
    for r in range(NREF):
        rx = jnp.floor(jnp.clip(xpix + off[:, 2 * r:2 * r + 1], 0.0, W - 1.0))
        ry = jnp.floor(jnp.clip(ypix + off[:, 2 * r + 1:2 * r + 2], 0.0, H - 1.0))
        cols.append(b * HW + ry.astype(jnp.int32) * W + rx.astype(jnp.int32))
    gidx_ref[...] = jnp.concatenate(cols, axis=1)


_tc_call = pl.pallas_call(
    _tc_body,
    grid=(B,),
    in_specs=[
        pl.BlockSpec((HW, C), lambda i: (i, 0)),
        pl.BlockSpec((C, C), lambda i: (0, 0)),
        pl.BlockSpec((C, C), lambda i: (0, 0)),
        pl.BlockSpec((C, C), lambda i: (0, 0)),
        pl.BlockSpec((C, 2 * NREF), lambda i: (0, 0)),
        pl.BlockSpec((1, C), lambda i: (0, 0)),
        pl.BlockSpec((1, C), lambda i: (0, 0)),
        pl.BlockSpec((1, C), lambda i: (0, 0)),
        pl.BlockSpec((1, 2 * NREF), lambda i: (0, 0)),
    ],
    out_specs=[
        pl.BlockSpec((HW, C), lambda i: (i, 0)),
        pl.BlockSpec((HW, HW), lambda i: (i, 0)),
        pl.BlockSpec((HW, NREF), lambda i: (i, 0)),
    ],
    out_shape=[
        jax.ShapeDtypeStruct((NPIX, C), jnp.float32),
        jax.ShapeDtypeStruct((NPIX, HW), jnp.float32),
        jax.ShapeDtypeStruct((NPIX, NREF), jnp.int32),
    ],
)


def _lane_splat(vec, lane):
    perm = jnp.broadcast_to(lane, (LANES,))
    return lax.gather(
        vec, perm[:, None],
        lax.GatherDimensionNumbers(offset_dims=(), collapsed_slice_dims=(0,),
                                   start_index_map=(0,)),
        slice_sizes=(1,), mode=lax.GatherScatterMode.PROMISE_IN_BOUNDS)


def _sc_body(v2, s2, gidxf, out2, idx_v, vrows, s_v, out_v, sem_in, sem_out):
    wid = lax.axis_index("s") * NC + lax.axis_index("c")

    def issue(gg, b):
        grp = wid * GPW + gg
        base = grp * GROUP
        pltpu.sync_copy(gidxf.at[pl.ds(grp * GROUP * NREF, GROUP * NREF)],
                        idx_v.at[b])
        pltpu.async_copy(v2.at[idx_v.at[b]], vrows.at[b], sem_in)
        for p in range(GROUP):
            pltpu.async_copy(s2.at[base + p],
                             s_v.at[b, pl.ds(p * HW, HW)], sem_in)

    def wait_in(b):
        pltpu.make_async_copy(v2.at[idx_v.at[b]], vrows.at[b], sem_in).wait()
        for p in range(GROUP):
            pltpu.make_async_copy(s2.at[0], s_v.at[b, pl.ds(p * HW, HW)],
                                  sem_in).wait()

    def drain_out(b):
        pltpu.make_async_copy(out_v.at[b], out2.at[pl.ds(0, GROUP)],
                              sem_out).wait()

    issue(0, 0)

    def group(g, _):
        buf = g & 1
        grp = wid * GPW + g
        base = grp * GROUP
        wait_in(buf)

        @pl.when(g + 1 < GPW)
        def _():
            issue(g + 1, 1 - buf)

        @pl.when(g >= 2)
        def _():
            drain_out(buf)

        chunks = [idx_v[buf, pl.ds(c * LANES, LANES)]
                  for c in range(GROUP * NREF // LANES)]
        for p in range(GROUP):
            avs = []
            for r in range(NREF):
                j = p * NREF + r
                li = chunks[j // LANES][j % LANES] & (HW - 1)
                start = pl.multiple_of(p * HW + (li & ~(LANES - 1)), LANES)
                cvec = s_v[buf, pl.ds(start, LANES)]
                zv = _lane_splat(cvec, li & (LANES - 1)) * SCALE
                avs.append(1.0 / (1.0 + jnp.exp(-zv)))
            j0 = p * NREF
            for cc in range(NCHUNK):
                sl = pl.ds(cc * LANES, LANES)
                o = avs[0] * vrows[buf, j0, sl]
                for r in range(1, NREF):
                    o = o + avs[r] * vrows[buf, j0 + r, sl]
                out_v[buf, p, sl] = o
        pltpu.async_copy(out_v.at[buf], out2.at[pl.ds(base, GROUP)], sem_out)
        return 0

    lax.fori_loop(0, GPW, group, 0)
    drain_out(0)
    drain_out(1)


@functools.cache
def _sc_call():
    return pl.kernel(
        _sc_body,
        out_type=jax.ShapeDtypeStruct((NPIX, C), jnp.float32),
        mesh=plsc.VectorSubcoreMesh(core_axis_name="c", subcore_axis_name="s"),
        scratch_types=[
            pltpu.VMEM((2, GROUP * NREF), jnp.int32),
            pltpu.VMEM((2, GROUP * NREF, C), jnp.float32),
            pltpu.VMEM((2, GROUP * HW), jnp.float32),
            pltpu.VMEM((2, GROUP, C), jnp.float32),
            pltpu.SemaphoreType.DMA,
            pltpu.SemaphoreType.DMA,
        ],
    )


def kernel(x, Wq, bq, Wk, bk, Wv, bv, Wo, bo):
    x2 = x.reshape(B, C, HW).transpose(0, 2, 1).reshape(NPIX, C)
    v2, s2, gidx = _tc_call(x2, Wq.T, Wk.T, Wv.T, Wo.T, bq[None, :],
                            bk[None, :], bv[None, :], bo[None, :])
    out2 = _sc_call()(v2, s2, gidx.reshape(NPIX * NREF))
    return out2.reshape(B, HW, C).transpose(0, 2, 1).reshape(B, C, H, W)


# speedup vs baseline: 1.9406x; 1.9406x over previous
import functools

import jax
import jax.numpy as jnp
from jax import lax
from jax.experimental import pallas as pl
from jax.experimental.pallas import tpu as pltpu
from jax.experimental.pallas import tpu_sc as plsc

B, C, H, W = 8, 768, 32, 32
HW = H * W
NPIX = B * HW
NREF = 4
LANES = 16
NC, NS = 2, 16
NW = NC * NS
PPW = NPIX // NW
GROUP = 8
GPW = PPW // GROUP
NCHUNK = C // LANES
SCALE = 1.0 / float(C) ** 0.5


def _tc_body(x_ref, wq_ref, wk_ref, wv_ref, wo_ref, bq_ref, bk_ref, bv_ref,
             bo_ref, v_ref, s_ref, gidx_ref):
    b = pl.program_id(0)
    xb = x_ref[...]
    q = jnp.dot(xb, wq_ref[...], preferred_element_type=jnp.float32) + bq_ref[...]
    k = jnp.dot(xb, wk_ref[...], preferred_element_type=jnp.float32) + bk_ref[...]
    v_ref[...] = jnp.dot(xb, wv_ref[...], preferred_element_type=jnp.float32) + bv_ref[...]
    s_ref[...] = lax.dot_general(q, k, (((1,), (1,)), ((), ())),
                                 preferred_element_type=jnp.float32)
    off = jnp.dot(q, wo_ref[...], preferred_element_type=jnp.float32) + bo_ref[...]
    p = lax.broadcasted_iota(jnp.int32, (HW, 1), 0)
    ypix = (p // W).astype(jnp.float32)
    xpix = (p % W).astype(jnp.float32)
    cols = []
    for r in range(NREF):
        rx = jnp.floor(jnp.clip(xpix + off[:, 2 * r:2 * r + 1], 0.0, W - 1.0))
        ry = jnp.floor(jnp.clip(ypix + off[:, 2 * r + 1:2 * r + 2], 0.0, H - 1.0))
        cols.append(b * HW + ry.astype(jnp.int32) * W + rx.astype(jnp.int32))
    gidx_ref[...] = jnp.concatenate(cols, axis=1)


_tc_call = pl.pallas_call(
    _tc_body,
    grid=(B,),
    in_specs=[
        pl.BlockSpec((HW, C), lambda i: (i, 0)),
        pl.BlockSpec((C, C), lambda i: (0, 0)),
        pl.BlockSpec((C, C), lambda i: (0, 0)),
        pl.BlockSpec((C, C), lambda i: (0, 0)),
        pl.BlockSpec((C, 2 * NREF), lambda i: (0, 0)),
        pl.BlockSpec((1, C), lambda i: (0, 0)),
        pl.BlockSpec((1, C), lambda i: (0, 0)),
        pl.BlockSpec((1, C), lambda i: (0, 0)),
        pl.BlockSpec((1, 2 * NREF), lambda i: (0, 0)),
    ],
    out_specs=[
        pl.BlockSpec((HW, C), lambda i: (i, 0)),
        pl.BlockSpec((HW, HW), lambda i: (i, 0)),
        pl.BlockSpec((HW, NREF), lambda i: (i, 0)),
    ],
    out_shape=[
        jax.ShapeDtypeStruct((NPIX, C), jnp.float32),
        jax.ShapeDtypeStruct((NPIX, HW), jnp.float32),
        jax.ShapeDtypeStruct((NPIX, NREF), jnp.int32),
    ],
)


def _lane_splat(vec, lane):
    perm = jnp.broadcast_to(lane, (LANES,))
    return lax.gather(
        vec, perm[:, None],
        lax.GatherDimensionNumbers(offset_dims=(), collapsed_slice_dims=(0,),
                                   start_index_map=(0,)),
        slice_sizes=(1,), mode=lax.GatherScatterMode.PROMISE_IN_BOUNDS)


def _sc_body(v2, s2, gidxf, out2, idx_v, sidx_v, vrows, s_v, out_v,
             sem_in, sem_out):
    wid = lax.axis_index("s") * NC + lax.axis_index("c")
    g0 = wid * GPW

    def issue(grp, b):
        base = grp * GROUP
        pltpu.sync_copy(gidxf.at[pl.ds(grp * (GROUP * NREF), GROUP * NREF)],
                        idx_v.at[b])
        pltpu.async_copy(v2.at[idx_v.at[b]], vrows.at[b], sem_in)
        sidx_v[b, pl.ds(0, LANES)] = (jnp.broadcast_to(base, (LANES,))
                                      + lax.iota(jnp.int32, LANES))
        pltpu.async_copy(s2.at[sidx_v.at[b, pl.ds(0, GROUP)]], s_v.at[b],
                         sem_in)

    def wait_in(b):
        pltpu.make_async_copy(v2.at[idx_v.at[b]], vrows.at[b], sem_in).wait()
        pltpu.make_async_copy(s2.at[sidx_v.at[b, pl.ds(0, GROUP)]],
                              s_v.at[b], sem_in).wait()

    def drain_out(b):
        pltpu.make_async_copy(out_v.at[b], out2.at[pl.ds(0, GROUP)],
                              sem_out).wait()

    issue(g0, 0)

    def pair(gp, _):
        for b in range(2):
            g = gp * 2 + b
            grp = g0 + g
            base = grp * GROUP
            wait_in(b)

            @pl.when(g + 1 < GPW)
            def _():
                issue(grp + 1, 1 - b)

            @pl.when(g >= 2)
            def _():
                drain_out(b)

            chunks = [idx_v[b, pl.ds(c * LANES, LANES)] for c in range(2)]
            for p in range(GROUP):
                avs = []
                for r in range(NREF):
                    j = p * NREF + r
                    li = chunks[j // LANES][j % LANES] & (HW - 1)
                    start = pl.multiple_of(li & ~(LANES - 1), LANES)
                    cvec = s_v[b, p, pl.ds(start, LANES)]
                    zv = _lane_splat(cvec, li & (LANES - 1)) * SCALE
                    avs.append(1.0 / (1.0 + jnp.exp(-zv)))
                j0 = p * NREF

                def wchunk(c8, _, b=b, p=p, j0=j0, avs=avs):
                    for u in range(8):
                        sl = pl.ds(pl.multiple_of(c8 * (8 * LANES) + u * LANES,
                                                  LANES), LANES)
                        o = avs[0] * vrows[b, j0, sl]
                        for r in range(1, NREF):
                            o = o + avs[r] * vrows[b, j0 + r, sl]
                        out_v[b, p, sl] = o
                    return 0

                lax.fori_loop(0, NCHUNK // 8, wchunk, 0)
            pltpu.async_copy(out_v.at[b], out2.at[pl.ds(base, GROUP)], sem_out)
        return 0

    lax.fori_loop(0, GPW // 2, pair, 0)
    drain_out(0)
    drain_out(1)


@functools.cache
def _sc_call():
    return pl.kernel(
        _sc_body,
        out_type=jax.ShapeDtypeStruct((NPIX, C), jnp.float32),
        mesh=plsc.VectorSubcoreMesh(core_axis_name="c", subcore_axis_name="s"),
        scratch_types=[
            pltpu.VMEM((2, GROUP * NREF), jnp.int32),
            pltpu.VMEM((2, LANES), jnp.int32),
            pltpu.VMEM((2, GROUP * NREF, C), jnp.float32),
            pltpu.VMEM((2, GROUP, HW), jnp.float32),
            pltpu.VMEM((2, GROUP, C), jnp.float32),
            pltpu.SemaphoreType.DMA,
            pltpu.SemaphoreType.DMA,
        ],
    )


def kernel(x, Wq, bq, Wk, bk, Wv, bv, Wo, bo):
    x2 = x.reshape(B, C, HW).transpose(0, 2, 1).reshape(NPIX, C)
    v2, s2, gidx = _tc_call(x2, Wq.T, Wk.T, Wv.T, Wo.T, bq[None, :],
                            bk[None, :], bv[None, :], bo[None, :])
    out2 = _sc_call()(v2, s2, gidx.reshape(NPIX * NREF))
    return out2.reshape(B, HW, C).transpose(0, 2, 1).reshape(B, C, H, W)
